# baseline (speedup 1.0000x reference)
import jax
import jax.numpy as jnp
from jax.experimental import pallas as pl

N_CTR = 128
KNN = 32


def _fps(xyz, n_centers):
    B, N, _ = xyz.shape
    def body(i, state):
        centroids, distances, farthest = state
        centroids = centroids.at[:, i].set(farthest)
        center = xyz[jnp.arange(B), farthest, :][:, None, :]
        dist = jnp.sum((xyz - center) ** 2, axis=-1)
        distances = jnp.minimum(distances, dist)
        farthest = jnp.argmax(distances, axis=-1).astype(jnp.int32)
        return (centroids, distances, farthest)
    centroids = jnp.zeros((B, n_centers), dtype=jnp.int32)
    distances = jnp.full((B, N), jnp.inf, dtype=jnp.float32)
    farthest = jnp.zeros((B,), dtype=jnp.int32)
    centroids, _, _ = jax.lax.fori_loop(0, n_centers, body, (centroids, distances, farthest))
    return centroids


def _bn(x, gamma, beta):
    mean = jnp.mean(x, axis=0, keepdims=True)
    var = jnp.var(x, axis=0, keepdims=True)
    return gamma * (x - mean) / jnp.sqrt(var + 1e-5) + beta


def _touch_kernel(x_ref, o_ref):
    o_ref[...] = x_ref[...] * 1.0


def kernel(xyz, W1, b1, g1, be1, W2, b2, g2, be2, W3, b3, g3, be3):
    B, N, _ = xyz.shape
    M = N_CTR
    center_idx = _fps(jax.lax.stop_gradient(xyz), N_CTR)
    centers = jnp.take_along_axis(xyz, jnp.broadcast_to(center_idx[..., None], (B, M, 3)), axis=1)
    dists = jnp.sum((centers[:, :, None, :] - xyz[:, None, :, :]) ** 2, axis=-1)
    neg_d, idx = jax.lax.top_k(-dists, KNN)
    xyz_e = jnp.broadcast_to(xyz[:, None, :, :], (B, M, N, 3))
    idx_e = jnp.broadcast_to(idx[..., None], (B, M, KNN, 3))
    patches = jnp.take_along_axis(xyz_e, idx_e, axis=2)
    patches = patches - centers[:, :, None, :]
    x = patches.reshape(B * M * KNN, 3)
    x = jax.nn.relu(_bn(x @ W1 + b1, g1, be1))
    x = jax.nn.relu(_bn(x @ W2 + b2, g2, be2))
    x = jax.nn.relu(_bn(x @ W3 + b3, g3, be3))
    x = x.reshape(B * M, KNN, -1)
    tokens = jnp.max(x, axis=1).reshape(B, M, -1)
    tokens = pl.pallas_call(
        _touch_kernel,
        out_shape=jax.ShapeDtypeStruct(tokens.shape, tokens.dtype),
    )(tokens)
    return (tokens, centers)


# speedup vs baseline: 6.1173x; 6.1173x over previous
import jax
import jax.numpy as jnp
from jax.experimental import pallas as pl

B = 16
N = 4096
M = 128
KNN = 32
ROWS = B * M * KNN


def _fps_body(x_ref, y_ref, z_ref, cx_ref, cy_ref, cz_ref):
    x = x_ref[...]
    y = y_ref[...]
    z = z_ref[...]
    pt = jax.lax.broadcasted_iota(jnp.int32, (B, N), 1)

    def body(i, carry):
        dist, far = carry
        cm = pt == far[:, None]
        cx = jnp.sum(jnp.where(cm, x, 0.0), axis=1)
        cy = jnp.sum(jnp.where(cm, y, 0.0), axis=1)
        cz = jnp.sum(jnp.where(cm, z, 0.0), axis=1)
        cx_ref[pl.ds(i, 1), :] = cx[None, :]
        cy_ref[pl.ds(i, 1), :] = cy[None, :]
        cz_ref[pl.ds(i, 1), :] = cz[None, :]
        dx = x - cx[:, None]
        dy = y - cy[:, None]
        dz = z - cz[:, None]
        d = dx * dx + dy * dy + dz * dz
        dist = jnp.minimum(dist, d)
        rm = jnp.max(dist, axis=1)
        far = jnp.min(jnp.where(dist == rm[:, None], pt, N), axis=1).astype(jnp.int32)
        return dist, far

    dist0 = jnp.full((B, N), jnp.inf, dtype=jnp.float32)
    far0 = jnp.zeros((B,), dtype=jnp.int32)
    jax.lax.fori_loop(0, M, body, (dist0, far0))


def _run_fps(xt):
    out = pl.pallas_call(
        _fps_body,
        out_shape=[jax.ShapeDtypeStruct((M, B), jnp.float32)] * 3,
    )(xt[0], xt[1], xt[2])
    return out


def _knn_body(x_ref, y_ref, z_ref, cx_ref, cy_ref, cz_ref,
              px_ref, py_ref, pz_ref):
    x = x_ref[0]
    y = y_ref[0]
    z = z_ref[0]
    cx = cx_ref[0]
    cy = cy_ref[0]
    cz = cz_ref[0]
    dxx = cx - x
    dyy = cy - y
    dzz = cz - z
    D = dxx * dxx + dyy * dyy + dzz * dzz
    pt = jax.lax.broadcasted_iota(jnp.int32, (M, N), 1)
    kio = jax.lax.broadcasted_iota(jnp.int32, (M, KNN), 1)
    inf = jnp.float32(jnp.inf)

    def body(r, carry):
        D, pxa, pya, pza = carry
        rm = jnp.min(D, axis=1, keepdims=True)
        ai = jnp.min(jnp.where(D == rm, pt, N), axis=1, keepdims=True)
        sel = pt == ai
        gx = jnp.sum(jnp.where(sel, x, 0.0), axis=1)
        gy = jnp.sum(jnp.where(sel, y, 0.0), axis=1)
        gz = jnp.sum(jnp.where(sel, z, 0.0), axis=1)
        hit = kio == r
        pxa = jnp.where(hit, (gx - cx[:, 0])[:, None], pxa)
        pya = jnp.where(hit, (gy - cy[:, 0])[:, None], pya)
        pza = jnp.where(hit, (gz - cz[:, 0])[:, None], pza)
        D = jnp.where(sel, inf, D)
        return D, pxa, pya, pza

    z0 = jnp.zeros((M, KNN), jnp.float32)
    _, pxa, pya, pza = jax.lax.fori_loop(0, KNN, body, (D, z0, z0, z0))
    px_ref[0] = pxa
    py_ref[0] = pya
    pz_ref[0] = pza


def _run_knn(xt, cxt):
    grid = (B,)
    pspec = pl.BlockSpec((1, 1, N), lambda b: (b, 0, 0))
    cspec = pl.BlockSpec((1, M, 1), lambda b: (b, 0, 0))
    ospec = pl.BlockSpec((1, M, KNN), lambda b: (b, 0, 0))
    out = pl.pallas_call(
        _knn_body,
        grid=grid,
        in_specs=[pspec] * 3 + [cspec] * 3,
        out_specs=[ospec] * 3,
        out_shape=[jax.ShapeDtypeStruct((B, M, KNN), jnp.float32)] * 3,
    )(xt[0], xt[1], xt[2], cxt[0], cxt[1], cxt[2])
    return out


G1 = 32
GR = (B * M) // G1

def _mlp1_body(px_ref, py_ref, pz_ref, w_ref, b_ref, z_ref, s_ref, q_ref):
    px = px_ref[...]
    py = py_ref[...]
    pz = pz_ref[...]
    wx = w_ref[0]
    wy = w_ref[1]
    wz = w_ref[2]
    b = b_ref[...]
    z = (px[:, :, None] * wx[None, None, :]
         + py[:, :, None] * wy[None, None, :]
         + pz[:, :, None] * wz[None, None, :]
         + b[None, :, :])
    z_ref[...] = z

    @pl.when(pl.program_id(0) == 0)
    def _():
        s_ref[...] = jnp.zeros_like(s_ref)
        q_ref[...] = jnp.zeros_like(q_ref)

    s_ref[...] += jnp.sum(z, axis=(0, 1))[None, :]
    q_ref[...] += jnp.sum(z * z, axis=(0, 1))[None, :]


def _run_mlp1(px2, py2, pz2, W1, b1):
    grid = (G1,)
    pspec = pl.BlockSpec((GR, KNN), lambda g: (g, 0))
    wspec = pl.BlockSpec((3, 64), lambda g: (0, 0))
    bspec = pl.BlockSpec((1, 64), lambda g: (0, 0))
    zspec = pl.BlockSpec((GR, KNN, 64), lambda g: (g, 0, 0))
    sspec = pl.BlockSpec((1, 64), lambda g: (0, 0))
    z1, s1, q1 = pl.pallas_call(
        _mlp1_body,
        grid=grid,
        in_specs=[pspec] * 3 + [wspec, bspec],
        out_specs=[zspec, sspec, sspec],
        out_shape=[
            jax.ShapeDtypeStruct((B * M, KNN, 64), jnp.float32),
            jax.ShapeDtypeStruct((1, 64), jnp.float32),
            jax.ShapeDtypeStruct((1, 64), jnp.float32),
        ],
    )(px2, py2, pz2, W1, b1)
    return z1, s1, q1


G2 = 16
R2 = ROWS // G2

def _mlp2_body(z1_ref, s_ref, h_ref, w_ref, b_ref, z2_ref, s2_ref, q2_ref):
    a = jnp.maximum(z1_ref[...] * s_ref[...] + h_ref[...], 0.0)
    z2 = jnp.dot(a, w_ref[...], preferred_element_type=jnp.float32) + b_ref[...]
    z2_ref[...] = z2

    @pl.when(pl.program_id(0) == 0)
    def _():
        s2_ref[...] = jnp.zeros_like(s2_ref)
        q2_ref[...] = jnp.zeros_like(q2_ref)

    s2_ref[...] += jnp.sum(z2, axis=0)[None, :]
    q2_ref[...] += jnp.sum(z2 * z2, axis=0)[None, :]


def _run_mlp2(z1f, sc1, sh1, W2, b2):
    grid = (G2,)
    z2, s2, q2 = pl.pallas_call(
        _mlp2_body,
        grid=grid,
        in_specs=[
            pl.BlockSpec((R2, 64), lambda g: (g, 0)),
            pl.BlockSpec((1, 64), lambda g: (0, 0)),
            pl.BlockSpec((1, 64), lambda g: (0, 0)),
            pl.BlockSpec((64, 128), lambda g: (0, 0)),
            pl.BlockSpec((1, 128), lambda g: (0, 0)),
        ],
        out_specs=[
            pl.BlockSpec((R2, 128), lambda g: (g, 0)),
            pl.BlockSpec((1, 128), lambda g: (0, 0)),
            pl.BlockSpec((1, 128), lambda g: (0, 0)),
        ],
        out_shape=[
            jax.ShapeDtypeStruct((ROWS, 128), jnp.float32),
            jax.ShapeDtypeStruct((1, 128), jnp.float32),
            jax.ShapeDtypeStruct((1, 128), jnp.float32),
        ],
    )(z1f, sc1, sh1, W2, b2)
    return z2, s2, q2


def _mlp3_body(z2_ref, s_ref, h_ref, w_ref, b_ref, a2_ref, s3_ref, q3_ref):
    a2 = jnp.maximum(z2_ref[...] * s_ref[...] + h_ref[...], 0.0)
    a2_ref[...] = a2
    z3 = jnp.dot(a2, w_ref[...], preferred_element_type=jnp.float32) + b_ref[...]

    @pl.when(pl.program_id(0) == 0)
    def _():
        s3_ref[...] = jnp.zeros_like(s3_ref)
        q3_ref[...] = jnp.zeros_like(q3_ref)

    s3_ref[...] += jnp.sum(z3, axis=0)[None, :]
    q3_ref[...] += jnp.sum(z3 * z3, axis=0)[None, :]


def _run_mlp3(z2, sc2, sh2, W3, b3):
    grid = (G2,)
    a2, s3, q3 = pl.pallas_call(
        _mlp3_body,
        grid=grid,
        in_specs=[
            pl.BlockSpec((R2, 128), lambda g: (g, 0)),
            pl.BlockSpec((1, 128), lambda g: (0, 0)),
            pl.BlockSpec((1, 128), lambda g: (0, 0)),
            pl.BlockSpec((128, 384), lambda g: (0, 0)),
            pl.BlockSpec((1, 384), lambda g: (0, 0)),
        ],
        out_specs=[
            pl.BlockSpec((R2, 128), lambda g: (g, 0)),
            pl.BlockSpec((1, 384), lambda g: (0, 0)),
            pl.BlockSpec((1, 384), lambda g: (0, 0)),
        ],
        out_shape=[
            jax.ShapeDtypeStruct((ROWS, 128), jnp.float32),
            jax.ShapeDtypeStruct((1, 384), jnp.float32),
            jax.ShapeDtypeStruct((1, 384), jnp.float32),
        ],
    )(z2, sc2, sh2, W3, b3)
    return a2, s3, q3


def _mlp4_body(a2_ref, s_ref, h_ref, w_ref, b_ref, t_ref):
    a2 = a2_ref[...]
    z3 = jnp.dot(a2, w_ref[...], preferred_element_type=jnp.float32) + b_ref[...]
    y = jnp.maximum(z3 * s_ref[...] + h_ref[...], 0.0)
    y = y.reshape(R2 // KNN, KNN, 384)
    t_ref[...] = jnp.max(y, axis=1)


def _run_mlp4(a2, sc3, sh3, W3, b3):
    grid = (G2,)
    toks = pl.pallas_call(
        _mlp4_body,
        grid=grid,
        in_specs=[
            pl.BlockSpec((R2, 128), lambda g: (g, 0)),
            pl.BlockSpec((1, 384), lambda g: (0, 0)),
            pl.BlockSpec((1, 384), lambda g: (0, 0)),
            pl.BlockSpec((128, 384), lambda g: (0, 0)),
            pl.BlockSpec((1, 384), lambda g: (0, 0)),
        ],
        out_specs=pl.BlockSpec((R2 // KNN, 384), lambda g: (g, 0)),
        out_shape=jax.ShapeDtypeStruct((B * M, 384), jnp.float32),
    )(a2, sc3, sh3, W3, b3)
    return toks


def _bn_coeffs(s, q, g, be):
    mean = s / ROWS
    var = q / ROWS - mean * mean
    sc = g[None, :] / jnp.sqrt(var + 1e-5)
    sh = be[None, :] - mean * sc
    return sc, sh


def kernel(xyz, W1, b1, g1, be1, W2, b2, g2, be2, W3, b3, g3, be3):
    xt = jnp.transpose(xyz, (2, 0, 1))
    cx, cy, cz = _run_fps(xt)
    centers = jnp.stack([cx.T, cy.T, cz.T], axis=-1)
    cxt = jnp.stack([cx.T, cy.T, cz.T])[:, :, :, None]
    px, py, pz = _run_knn(xt[:, :, None, :], cxt)
    px2 = px.reshape(B * M, KNN)
    py2 = py.reshape(B * M, KNN)
    pz2 = pz.reshape(B * M, KNN)
    z1, s1, q1 = _run_mlp1(px2, py2, pz2, W1, b1[None, :])
    sc1, sh1 = _bn_coeffs(s1, q1, g1, be1)
    z1f = z1.reshape(ROWS, 64)
    z2, s2, q2 = _run_mlp2(z1f, sc1, sh1, W2, b2[None, :])
    sc2, sh2 = _bn_coeffs(s2, q2, g2, be2)
    a2, s3, q3 = _run_mlp3(z2, sc2, sh2, W3, b3[None, :])
    sc3, sh3 = _bn_coeffs(s3, q3, g3, be3)
    toks = _run_mlp4(a2, sc3, sh3, W3, b3[None, :])
    tokens = toks.reshape(B, M, 384)
    return (tokens, centers)
